# baseline (speedup 1.0000x reference)
import functools

import jax
import jax.numpy as jnp
from jax.experimental import pallas as pl
from jax.experimental.pallas import tpu as pltpu

_NN = 9
_LANES = 128
_SUB = 1024

_SORT8 = (
    (0, 1), (2, 3), (4, 5), (6, 7),
    (0, 2), (1, 3), (4, 6), (5, 7),
    (1, 2), (5, 6),
    (0, 4), (1, 5), (2, 6), (3, 7),
    (2, 4), (3, 5),
    (1, 2), (3, 4), (5, 6),
)
_MERGE9 = (
    (0, 8),
    (1, 5), (2, 6), (3, 7), (4, 8),
    (1, 3), (2, 4), (5, 7), (6, 8),
    (1, 2), (3, 4), (5, 6), (7, 8),
)


def _merge_group(rows, vs):
    vs = list(vs)
    for i, j in _SORT8:
        lo = jnp.minimum(vs[i], vs[j])
        vs[j] = jnp.maximum(vs[i], vs[j])
        vs[i] = lo
    cand = [rows[0]] + [
        jnp.minimum(rows[i], vs[8 - i]) for i in range(1, _NN)
    ]
    for i, j in _MERGE9:
        lo = jnp.minimum(cand[i], cand[j])
        cand[j] = jnp.maximum(cand[i], cand[j])
        cand[i] = lo
    return cand


def _topk_body(x_ref, yt_ref, out_ref, r_scr, *, nk, qt, kb):
    k = pl.program_id(1)

    @pl.when(k == 0)
    def _():
        r_scr[...] = jnp.full((qt, _NN * _LANES), jnp.inf, jnp.float32)

    yb = yt_ref[...]
    d = yb.shape[0] // 2
    yf = yb[:d].astype(jnp.float32) + yb[d:].astype(jnp.float32)
    yn = jnp.sum(yf * yf, axis=0, keepdims=True)
    xs = x_ref[...] * (-2.0)
    xh = xs.astype(jnp.bfloat16)
    xcat = jnp.concatenate([xh, xh], axis=1)
    dims = (((1,), (0,)), ((), ()))

    rows = [r_scr[:, j * _LANES:(j + 1) * _LANES] for j in range(_NN)]

    g_prev = None
    for sub in range(kb // _SUB):
        c0 = sub * _SUB
        s = jax.lax.dot_general(
            xcat,
            yb[:, c0:c0 + _SUB],
            dims,
            precision=jax.lax.Precision.DEFAULT,
            preferred_element_type=jnp.float32,
        )
        g = s + yn[:, c0:c0 + _SUB]
        if g_prev is not None:
            for grp in range(_SUB // (8 * _LANES)):
                rows = _merge_group(rows, [
                    g_prev[:, (grp * 8 + t) * _LANES:(grp * 8 + t + 1) * _LANES]
                    for t in range(8)
                ])
        g_prev = g
    for grp in range(_SUB // (8 * _LANES)):
        rows = _merge_group(rows, [
            g_prev[:, (grp * 8 + t) * _LANES:(grp * 8 + t + 1) * _LANES]
            for t in range(8)
        ])

    for j in range(_NN):
        r_scr[:, j * _LANES:(j + 1) * _LANES] = rows[j]

    @pl.when(k == nk - 1)
    def _():
        x = x_ref[...]
        xn = jnp.sum(x * x, axis=1, keepdims=True)
        d2 = jnp.maximum(r_scr[...] + xn, 0.0)
        bits = jax.lax.bitcast_convert_type(d2, jnp.int32)
        tag = jax.lax.broadcasted_iota(jnp.int32, (qt, _NN * _LANES), 1)
        f = jax.lax.bitcast_convert_type((bits & -2048) | tag, jnp.float32)
        cols = []
        for _ in range(_NN):
            m = jnp.min(f, axis=1, keepdims=True)
            cols.append(m)
            f = jnp.where(f == m, jnp.inf, f)
        vals = jnp.concatenate(cols, axis=1)
        vb = jax.lax.bitcast_convert_type(vals, jnp.int32) & -2048
        out_ref[...] = jnp.sqrt(jax.lax.bitcast_convert_type(vb, jnp.float32))


def _pick_qt(q):
    if q % 8 == 0:
        return q
    for qt in range(q, 7, -8):
        if q % qt == 0:
            return qt
    return q


@jax.jit
def kernel(embedding, memory_bank):
    q, d = embedding.shape
    k, _ = memory_bank.shape
    qt = _pick_qt(q)
    kb = 4096 if k % 4096 == 0 else k
    nq = q // qt
    nk = k // kb

    yh = memory_bank.astype(jnp.bfloat16)
    yl = (memory_bank - yh.astype(jnp.float32)).astype(jnp.bfloat16)
    ycat = jnp.concatenate([yh, yl], axis=1).T

    body = functools.partial(_topk_body, nk=nk, qt=qt, kb=kb)
    return pl.pallas_call(
        body,
        grid=(nq, nk),
        in_specs=[
            pl.BlockSpec((qt, d), lambda iq, ik: (iq, 0)),
            pl.BlockSpec((2 * d, kb), lambda iq, ik: (0, ik)),
        ],
        out_specs=pl.BlockSpec((qt, _NN), lambda iq, ik: (iq, 0)),
        out_shape=jax.ShapeDtypeStruct((q, _NN), jnp.float32),
        scratch_shapes=[
            pltpu.VMEM((qt, _NN * _LANES), jnp.float32),
        ],
    )(embedding, ycat)


# speedup vs baseline: 1.0036x; 1.0036x over previous
import functools

import jax
import jax.numpy as jnp
from jax.experimental import pallas as pl
from jax.experimental.pallas import tpu as pltpu

_NN = 9
_LANES = 128
_SUB = 1024

_SORT8 = (
    (0, 1), (2, 3), (4, 5), (6, 7),
    (0, 2), (1, 3), (4, 6), (5, 7),
    (1, 2), (5, 6),
    (0, 4), (1, 5), (2, 6), (3, 7),
    (2, 4), (3, 5),
    (1, 2), (3, 4), (5, 6),
)
_MERGE9 = (
    (0, 8),
    (1, 5), (2, 6), (3, 7), (4, 8),
    (1, 3), (2, 4), (5, 7), (6, 8),
    (1, 2), (3, 4), (5, 6), (7, 8),
)


def _merge_group(rows, vs):
    vs = list(vs)
    for i, j in _SORT8:
        lo = jnp.minimum(vs[i], vs[j])
        vs[j] = jnp.maximum(vs[i], vs[j])
        vs[i] = lo
    cand = [rows[0]] + [
        jnp.minimum(rows[i], vs[8 - i]) for i in range(1, _NN)
    ]
    for i, j in _MERGE9:
        lo = jnp.minimum(cand[i], cand[j])
        cand[j] = jnp.maximum(cand[i], cand[j])
        cand[i] = lo
    return cand


def _topk_body(x_ref, yt_ref, yc_ref, out_ref, r_scr, *, nk, qt, kb):
    k = pl.program_id(1)

    @pl.when(k == 0)
    def _():
        r_scr[...] = jnp.full((qt, _NN * _LANES), jnp.inf, jnp.float32)

    yb = yt_ref[...]
    yn = jnp.sum(yb * yb, axis=0, keepdims=True)
    yc = yc_ref[...]
    xs = x_ref[...] * (-2.0)
    xh = xs.astype(jnp.bfloat16)
    xcat = jnp.concatenate([xh, xh], axis=1)
    dims = (((1,), (0,)), ((), ()))

    rows = [r_scr[:, j * _LANES:(j + 1) * _LANES] for j in range(_NN)]

    g_prev = None
    for sub in range(kb // _SUB):
        c0 = sub * _SUB
        s = jax.lax.dot_general(
            xcat,
            yc[:, c0:c0 + _SUB],
            dims,
            precision=jax.lax.Precision.DEFAULT,
            preferred_element_type=jnp.float32,
        )
        g = s + yn[:, c0:c0 + _SUB]
        if g_prev is not None:
            for grp in range(_SUB // (8 * _LANES)):
                rows = _merge_group(rows, [
                    g_prev[:, (grp * 8 + t) * _LANES:(grp * 8 + t + 1) * _LANES]
                    for t in range(8)
                ])
        g_prev = g
    for grp in range(_SUB // (8 * _LANES)):
        rows = _merge_group(rows, [
            g_prev[:, (grp * 8 + t) * _LANES:(grp * 8 + t + 1) * _LANES]
            for t in range(8)
        ])

    for j in range(_NN):
        r_scr[:, j * _LANES:(j + 1) * _LANES] = rows[j]

    @pl.when(k == nk - 1)
    def _():
        x = x_ref[...]
        xn = jnp.sum(x * x, axis=1, keepdims=True)
        d2 = jnp.maximum(r_scr[...] + xn, 0.0)
        bits = jax.lax.bitcast_convert_type(d2, jnp.int32)
        tag = jax.lax.broadcasted_iota(jnp.int32, (qt, _NN * _LANES), 1)
        f = jax.lax.bitcast_convert_type((bits & -2048) | tag, jnp.float32)
        cols = []
        for _ in range(_NN):
            m = jnp.min(f, axis=1, keepdims=True)
            cols.append(m)
            f = jnp.where(f == m, jnp.inf, f)
        vals = jnp.concatenate(cols, axis=1)
        vb = jax.lax.bitcast_convert_type(vals, jnp.int32) & -2048
        out_ref[...] = jnp.sqrt(jax.lax.bitcast_convert_type(vb, jnp.float32))


def _pick_qt(q):
    if q % 8 == 0:
        return q
    for qt in range(q, 7, -8):
        if q % qt == 0:
            return qt
    return q


@jax.jit
def kernel(embedding, memory_bank):
    q, d = embedding.shape
    k, _ = memory_bank.shape
    qt = _pick_qt(q)
    kb = 4096 if k % 4096 == 0 else k
    nq = q // qt
    nk = k // kb

    yh = memory_bank.astype(jnp.bfloat16)
    yl = (memory_bank - yh.astype(jnp.float32)).astype(jnp.bfloat16)
    ycat = jnp.concatenate([yh, yl], axis=1).T

    body = functools.partial(_topk_body, nk=nk, qt=qt, kb=kb)
    return pl.pallas_call(
        body,
        grid=(nq, nk),
        in_specs=[
            pl.BlockSpec((qt, d), lambda iq, ik: (iq, 0)),
            pl.BlockSpec((d, kb), lambda iq, ik: (0, ik)),
            pl.BlockSpec((2 * d, kb), lambda iq, ik: (0, ik)),
        ],
        out_specs=pl.BlockSpec((qt, _NN), lambda iq, ik: (iq, 0)),
        out_shape=jax.ShapeDtypeStruct((q, _NN), jnp.float32),
        scratch_shapes=[
            pltpu.VMEM((qt, _NN * _LANES), jnp.float32),
        ],
    )(embedding, memory_bank.T, ycat)


# speedup vs baseline: 1.2518x; 1.2474x over previous
import functools

import jax
import jax.numpy as jnp
from jax.experimental import pallas as pl
from jax.experimental.pallas import tpu as pltpu

_NN = 9
_LANES = 128
_SUB = 1024

_SORT8 = (
    (0, 1), (2, 3), (4, 5), (6, 7),
    (0, 2), (1, 3), (4, 6), (5, 7),
    (1, 2), (5, 6),
    (0, 4), (1, 5), (2, 6), (3, 7),
    (2, 4), (3, 5),
    (1, 2), (3, 4), (5, 6),
)
_MERGE9 = (
    (0, 8),
    (1, 5), (2, 6), (3, 7), (4, 8),
    (1, 3), (2, 4), (5, 7), (6, 8),
    (1, 2), (3, 4), (5, 6), (7, 8),
)


def _merge_group(rows, vs):
    vs = list(vs)
    for i, j in _SORT8:
        lo = jnp.minimum(vs[i], vs[j])
        vs[j] = jnp.maximum(vs[i], vs[j])
        vs[i] = lo
    cand = [rows[0]] + [
        jnp.minimum(rows[i], vs[8 - i]) for i in range(1, _NN)
    ]
    for i, j in _MERGE9:
        lo = jnp.minimum(cand[i], cand[j])
        cand[j] = jnp.maximum(cand[i], cand[j])
        cand[i] = lo
    return cand


def _topk_body(x_ref, yt_ref, out_ref, r_scr, *, nk, qt, kb):
    k = pl.program_id(1)

    @pl.when(k == 0)
    def _():
        r_scr[...] = jnp.full((qt, _NN * _LANES), jnp.inf, jnp.float32)

    yb = yt_ref[...]
    yn = jnp.sum(yb * yb, axis=0, keepdims=True)
    xs = x_ref[...] * (-2.0)
    dims = (((1,), (0,)), ((), ()))

    rows = [r_scr[:, j * _LANES:(j + 1) * _LANES] for j in range(_NN)]

    g_prev = None
    for sub in range(kb // _SUB):
        c0 = sub * _SUB
        s = jax.lax.dot_general(
            xs,
            yb[:, c0:c0 + _SUB],
            dims,
            precision=jax.lax.Precision.DEFAULT,
            preferred_element_type=jnp.float32,
        )
        g = s + yn[:, c0:c0 + _SUB]
        if g_prev is not None:
            for grp in range(_SUB // (8 * _LANES)):
                rows = _merge_group(rows, [
                    g_prev[:, (grp * 8 + t) * _LANES:(grp * 8 + t + 1) * _LANES]
                    for t in range(8)
                ])
        g_prev = g
    for grp in range(_SUB // (8 * _LANES)):
        rows = _merge_group(rows, [
            g_prev[:, (grp * 8 + t) * _LANES:(grp * 8 + t + 1) * _LANES]
            for t in range(8)
        ])

    for j in range(_NN):
        r_scr[:, j * _LANES:(j + 1) * _LANES] = rows[j]

    @pl.when(k == nk - 1)
    def _():
        x = x_ref[...]
        xn = jnp.sum(x * x, axis=1, keepdims=True)
        d2 = jnp.maximum(r_scr[...] + xn, 0.0)
        bits = jax.lax.bitcast_convert_type(d2, jnp.int32)
        tag = jax.lax.broadcasted_iota(jnp.int32, (qt, _NN * _LANES), 1)
        f = jax.lax.bitcast_convert_type((bits & -2048) | tag, jnp.float32)
        cols = []
        for _ in range(_NN):
            m = jnp.min(f, axis=1, keepdims=True)
            cols.append(m)
            f = jnp.where(f == m, jnp.inf, f)
        vals = jnp.concatenate(cols, axis=1)
        vb = jax.lax.bitcast_convert_type(vals, jnp.int32) & -2048
        out_ref[...] = jnp.sqrt(jax.lax.bitcast_convert_type(vb, jnp.float32))


def _pick_qt(q):
    if q % 8 == 0:
        return q
    for qt in range(q, 7, -8):
        if q % qt == 0:
            return qt
    return q


@jax.jit
def kernel(embedding, memory_bank):
    q, d = embedding.shape
    k, _ = memory_bank.shape
    qt = _pick_qt(q)
    kb = 4096 if k % 4096 == 0 else k
    nq = q // qt
    nk = k // kb

    body = functools.partial(_topk_body, nk=nk, qt=qt, kb=kb)
    return pl.pallas_call(
        body,
        grid=(nq, nk),
        in_specs=[
            pl.BlockSpec((qt, d), lambda iq, ik: (iq, 0)),
            pl.BlockSpec((d, kb), lambda iq, ik: (0, ik)),
        ],
        out_specs=pl.BlockSpec((qt, _NN), lambda iq, ik: (iq, 0)),
        out_shape=jax.ShapeDtypeStruct((q, _NN), jnp.float32),
        scratch_shapes=[
            pltpu.VMEM((qt, _NN * _LANES), jnp.float32),
        ],
    )(embedding, memory_bank.T)


# speedup vs baseline: 1.2519x; 1.0001x over previous
import functools

import jax
import jax.numpy as jnp
from jax.experimental import pallas as pl
from jax.experimental.pallas import tpu as pltpu

_NN = 9
_LANES = 128
_SUB = 1024

_SORT8 = (
    (0, 1), (2, 3), (4, 5), (6, 7),
    (0, 2), (1, 3), (4, 6), (5, 7),
    (1, 2), (5, 6),
    (0, 4), (1, 5), (2, 6), (3, 7),
    (2, 4), (3, 5),
    (1, 2), (3, 4), (5, 6),
)
_MERGE9 = (
    (0, 8),
    (1, 5), (2, 6), (3, 7), (4, 8),
    (1, 3), (2, 4), (5, 7), (6, 8),
    (1, 2), (3, 4), (5, 6), (7, 8),
)


def _merge_group(rows, vs):
    vs = list(vs)
    for i, j in _SORT8:
        lo = jnp.minimum(vs[i], vs[j])
        vs[j] = jnp.maximum(vs[i], vs[j])
        vs[i] = lo
    cand = [rows[0]] + [
        jnp.minimum(rows[i], vs[8 - i]) for i in range(1, _NN)
    ]
    for i, j in _MERGE9:
        lo = jnp.minimum(cand[i], cand[j])
        cand[j] = jnp.maximum(cand[i], cand[j])
        cand[i] = lo
    return cand


def _topk_body(x_ref, yt_ref, out_ref, r_scr, *, nk, qt, kb):
    k = pl.program_id(1)

    @pl.when(k == 0)
    def _():
        r_scr[...] = jnp.full((qt, _NN * _LANES), jnp.inf, jnp.float32)

    yb = yt_ref[...]
    yn = jnp.sum(yb * yb, axis=0, keepdims=True)
    xs = x_ref[...] * (-2.0)
    dims = (((1,), (0,)), ((), ()))

    rows = [r_scr[:, j * _LANES:(j + 1) * _LANES] for j in range(_NN)]

    g_prev = None
    for sub in range(kb // _SUB):
        c0 = sub * _SUB
        s = jax.lax.dot_general(
            xs,
            yb[:, c0:c0 + _SUB],
            dims,
            precision=jax.lax.Precision.DEFAULT,
            preferred_element_type=jnp.float32,
        )
        g = s + yn[:, c0:c0 + _SUB]
        if g_prev is not None:
            for grp in range(_SUB // (8 * _LANES)):
                rows = _merge_group(rows, [
                    g_prev[:, (grp * 8 + t) * _LANES:(grp * 8 + t + 1) * _LANES]
                    for t in range(8)
                ])
        g_prev = g
    for grp in range(_SUB // (8 * _LANES)):
        rows = _merge_group(rows, [
            g_prev[:, (grp * 8 + t) * _LANES:(grp * 8 + t + 1) * _LANES]
            for t in range(8)
        ])

    for j in range(_NN):
        r_scr[:, j * _LANES:(j + 1) * _LANES] = rows[j]

    @pl.when(k == nk - 1)
    def _():
        x = x_ref[...]
        xn = jnp.sum(x * x, axis=1, keepdims=True)
        d2 = jnp.maximum(r_scr[...] + xn, 0.0)
        bits = jax.lax.bitcast_convert_type(d2, jnp.int32)
        tag = jax.lax.broadcasted_iota(jnp.int32, (qt, _NN * _LANES), 1)
        f = jax.lax.bitcast_convert_type((bits & -2048) | tag, jnp.float32)
        cols = []
        for _ in range(_NN):
            m = jnp.min(f, axis=1, keepdims=True)
            cols.append(m)
            f = jnp.where(f == m, jnp.inf, f)
        vals = jnp.concatenate(cols, axis=1)
        vb = jax.lax.bitcast_convert_type(vals, jnp.int32) & -2048
        out_ref[...] = jnp.sqrt(jax.lax.bitcast_convert_type(vb, jnp.float32))


def _pick_qt(q):
    if q % 8 == 0:
        return q
    for qt in range(q, 7, -8):
        if q % qt == 0:
            return qt
    return q


@jax.jit
def kernel(embedding, memory_bank):
    q, d = embedding.shape
    k, _ = memory_bank.shape
    qt = _pick_qt(q)
    kb = 8192 if k % 8192 == 0 else k
    nq = q // qt
    nk = k // kb

    body = functools.partial(_topk_body, nk=nk, qt=qt, kb=kb)
    return pl.pallas_call(
        body,
        grid=(nq, nk),
        in_specs=[
            pl.BlockSpec((qt, d), lambda iq, ik: (iq, 0)),
            pl.BlockSpec((d, kb), lambda iq, ik: (0, ik)),
        ],
        out_specs=pl.BlockSpec((qt, _NN), lambda iq, ik: (iq, 0)),
        out_shape=jax.ShapeDtypeStruct((q, _NN), jnp.float32),
        scratch_shapes=[
            pltpu.VMEM((qt, _NN * _LANES), jnp.float32),
        ],
    )(embedding, memory_bank.T)


# speedup vs baseline: 4.3040x; 3.4379x over previous
import functools

import jax
import jax.numpy as jnp
from jax.experimental import pallas as pl
from jax.experimental.pallas import tpu as pltpu

_NN = 9
_LANES = 128
_SUB = 512

_SORT8 = (
    (0, 1), (2, 3), (4, 5), (6, 7),
    (0, 2), (1, 3), (4, 6), (5, 7),
    (1, 2), (5, 6),
    (0, 4), (1, 5), (2, 6), (3, 7),
    (2, 4), (3, 5),
    (1, 2), (3, 4), (5, 6),
)
_MERGE9 = (
    (0, 8),
    (1, 5), (2, 6), (3, 7), (4, 8),
    (1, 3), (2, 4), (5, 7), (6, 8),
    (1, 2), (3, 4), (5, 6), (7, 8),
)


def _merge_group(rows, vs):
    vs = list(vs)
    for i, j in _SORT8:
        lo = jnp.minimum(vs[i], vs[j])
        vs[j] = jnp.maximum(vs[i], vs[j])
        vs[i] = lo
    cand = [rows[0]] + [
        jnp.minimum(rows[i], vs[8 - i]) for i in range(1, _NN)
    ]
    for i, j in _MERGE9:
        lo = jnp.minimum(cand[i], cand[j])
        cand[j] = jnp.maximum(cand[i], cand[j])
        cand[i] = lo
    return cand


def _topk_body(x_ref, yt_ref, out_ref, r_scr, *, nk, qt, kb):
    k = pl.program_id(1)

    @pl.when(k == 0)
    def _():
        r_scr[...] = jnp.full((qt, _NN * _LANES), jnp.inf, jnp.float32)

    yb = yt_ref[...]
    yn = jnp.sum(yb * yb, axis=0, keepdims=True)
    xs = x_ref[...] * (-2.0)
    dims = (((1,), (0,)), ((), ()))

    rows = [r_scr[:, j * _LANES:(j + 1) * _LANES] for j in range(_NN)]

    g_prev = None
    for sub in range(kb // _SUB):
        c0 = sub * _SUB
        s = jax.lax.dot_general(
            xs,
            yb[:, c0:c0 + _SUB],
            dims,
            precision=jax.lax.Precision.DEFAULT,
            preferred_element_type=jnp.float32,
        )
        g = s + yn[:, c0:c0 + _SUB]
        if g_prev is not None:
            for grp in range(_SUB // (8 * _LANES)):
                rows = _merge_group(rows, [
                    g_prev[:, (grp * 8 + t) * _LANES:(grp * 8 + t + 1) * _LANES]
                    for t in range(8)
                ])
        g_prev = g
    for grp in range(_SUB // (8 * _LANES)):
        rows = _merge_group(rows, [
            g_prev[:, (grp * 8 + t) * _LANES:(grp * 8 + t + 1) * _LANES]
            for t in range(8)
        ])

    for j in range(_NN):
        r_scr[:, j * _LANES:(j + 1) * _LANES] = rows[j]

    @pl.when(k == nk - 1)
    def _():
        x = x_ref[...]
        xn = jnp.sum(x * x, axis=1, keepdims=True)
        d2 = jnp.maximum(r_scr[...] + xn, 0.0)
        bits = jax.lax.bitcast_convert_type(d2, jnp.int32)
        tag = jax.lax.broadcasted_iota(jnp.int32, (qt, _NN * _LANES), 1)
        f = jax.lax.bitcast_convert_type((bits & -2048) | tag, jnp.float32)
        cols = []
        for _ in range(_NN):
            m = jnp.min(f, axis=1, keepdims=True)
            cols.append(m)
            f = jnp.where(f == m, jnp.inf, f)
        vals = jnp.concatenate(cols, axis=1)
        vb = jax.lax.bitcast_convert_type(vals, jnp.int32) & -2048
        out_ref[...] = jnp.sqrt(jax.lax.bitcast_convert_type(vb, jnp.float32))


def _pick_qt(q):
    if q % 8 == 0:
        return q
    for qt in range(q, 7, -8):
        if q % qt == 0:
            return qt
    return q


@jax.jit
def kernel(embedding, memory_bank):
    q, d = embedding.shape
    k, _ = memory_bank.shape
    qt = _pick_qt(q)
    kb = 8192 if k % 8192 == 0 else k
    nq = q // qt
    nk = k // kb

    body = functools.partial(_topk_body, nk=nk, qt=qt, kb=kb)
    return pl.pallas_call(
        body,
        grid=(nq, nk),
        in_specs=[
            pl.BlockSpec((qt, d), lambda iq, ik: (iq, 0)),
            pl.BlockSpec((d, kb), lambda iq, ik: (0, ik)),
        ],
        out_specs=pl.BlockSpec((qt, _NN), lambda iq, ik: (iq, 0)),
        out_shape=jax.ShapeDtypeStruct((q, _NN), jnp.float32),
        scratch_shapes=[
            pltpu.VMEM((qt, _NN * _LANES), jnp.float32),
        ],
    )(embedding, memory_bank.T)
